# baseline (speedup 1.0000x reference)
import functools

import jax
import jax.numpy as jnp
from jax import lax
from jax.experimental import pallas as pl
from jax.experimental.pallas import tpu as pltpu
from jax.experimental.pallas import tpu_sc as plsc

B = 2
N_IN = 12
KEEP = 3
ROW = 384 * 32 * 32
NW = 32
CHUNK = ROW // NW


def _sc_body(in_hbm, scores_hbm, out_a, out_b, scores_v, bufs, sem_in, sem_out):
    wid = lax.axis_index("s") * 2 + lax.axis_index("c")
    off = wid * CHUNK

    pltpu.sync_copy(scores_hbm, scores_v)

    src_rows = [jnp.int32(0)] * (B * KEEP)
    for b in range(B):
        s_vec = scores_v[b]
        s = [s_vec[i] for i in range(N_IN)]
        for j in range(N_IN):
            sj = s[j]
            rank = jnp.int32(0)
            for k in range(N_IN):
                sk = s[k]
                before = (sk < sj) | ((sk == sj) & (k < j))
                rank = rank + jnp.where(before, 1, 0)
            for p in range(KEEP):
                sel = rank == (KEEP + p)
                src_rows[b * KEEP + p] = jnp.where(
                    sel, jnp.int32(b * N_IN + j), src_rows[b * KEEP + p]
                )

    gathers = [
        pltpu.async_copy(in_hbm.at[src_rows[j], pl.ds(off, CHUNK)], bufs.at[j], sem_in)
        for j in range(B * KEEP)
    ]
    for g in gathers:
        g.wait()
    stores = []
    for j in range(B * KEEP):
        stores.append(pltpu.async_copy(bufs.at[j], out_a.at[j, pl.ds(off, CHUNK)], sem_out))
        stores.append(pltpu.async_copy(bufs.at[j], out_b.at[j, pl.ds(off, CHUNK)], sem_out))
    for s in stores:
        s.wait()


@jax.jit
def _sc_gather(flat_in, scores_padded):
    mesh = plsc.VectorSubcoreMesh(core_axis_name="c", subcore_axis_name="s")
    out_row = jax.ShapeDtypeStruct((B * KEEP, ROW), jnp.float32)
    f = pl.kernel(
        _sc_body,
        out_type=(out_row, out_row),
        mesh=mesh,
        scratch_types=[
            pltpu.VMEM((B, 16), jnp.float32),
            pltpu.VMEM((B * KEEP, CHUNK), jnp.float32),
            pltpu.SemaphoreType.DMA,
            pltpu.SemaphoreType.DMA,
        ],
    )
    return f(flat_in, scores_padded)


def kernel(image_latent):
    scores = jax.random.uniform(jax.random.key(42), (B, N_IN), dtype=jnp.float32)
    scores_padded = jnp.full((B, 16), 2.0, jnp.float32).at[:, :N_IN].set(scores)
    flat_in = image_latent.reshape(B * N_IN, ROW)
    out_a, out_b = _sc_gather(flat_in, scores_padded)
    return (
        out_a.reshape(B, KEEP, 384, 32, 32),
        out_b.reshape(B * KEEP, 384, 32, 32),
    )
